# baseline (speedup 1.0000x reference)
import functools

import jax
import jax.numpy as jnp
from jax.experimental import pallas as pl
from jax.experimental.pallas import tpu as pltpu

POS_DIM = 3
FEAT_DIM = 128
HIDDEN = 256
BLOCK_N = 2048


def _mlp_block_kernel(pos_ref, w_ref, w1_ref, b1_ref, w2p_ref, b2p_ref,
                      w2w_ref, b2w_ref, out_pos_ref, out_w_ref):
    w = w_ref[...]
    h = jnp.maximum(
        jnp.dot(w.astype(jnp.bfloat16), w1_ref[...],
                preferred_element_type=jnp.float32)
        + b1_ref[...], 0.0)
    hb = h.astype(jnp.bfloat16)
    dp = jnp.dot(hb, w2p_ref[...], preferred_element_type=jnp.float32)
    dw = jnp.dot(hb, w2w_ref[...], preferred_element_type=jnp.float32)
    out_pos_ref[...] = pos_ref[...] + dp + b2p_ref[...]
    out_w_ref[...] = w + dw + b2w_ref[...]


@functools.partial(jax.jit, static_argnames=())
def kernel(positions, weights, batch, W1, b1, W2, b2):
    del batch
    n = weights.shape[0]
    grid = (n // BLOCK_N,)

    W1 = W1.astype(jnp.bfloat16)
    W2p = W2[:, :POS_DIM].astype(jnp.bfloat16)
    W2w = W2[:, POS_DIM:].astype(jnp.bfloat16)
    b1r = b1.reshape(1, HIDDEN)
    b2p = b2[:POS_DIM].reshape(1, POS_DIM)
    b2w = b2[POS_DIM:].reshape(1, FEAT_DIM)

    row_block = lambda i: (i, 0)
    rep = lambda i: (0, 0)
    out_pos, out_w = pl.pallas_call(
        _mlp_block_kernel,
        grid=grid,
        in_specs=[
            pl.BlockSpec((BLOCK_N, POS_DIM), row_block),
            pl.BlockSpec((BLOCK_N, FEAT_DIM), row_block),
            pl.BlockSpec((FEAT_DIM, HIDDEN), rep),
            pl.BlockSpec((1, HIDDEN), rep),
            pl.BlockSpec((HIDDEN, POS_DIM), rep),
            pl.BlockSpec((1, POS_DIM), rep),
            pl.BlockSpec((HIDDEN, FEAT_DIM), rep),
            pl.BlockSpec((1, FEAT_DIM), rep),
        ],
        out_specs=[
            pl.BlockSpec((BLOCK_N, POS_DIM), row_block),
            pl.BlockSpec((BLOCK_N, FEAT_DIM), row_block),
        ],
        out_shape=[
            jax.ShapeDtypeStruct((n, POS_DIM), jnp.float32),
            jax.ShapeDtypeStruct((n, FEAT_DIM), jnp.float32),
        ],
        compiler_params=pltpu.CompilerParams(
            dimension_semantics=("arbitrary",),
        ),
    )(positions, weights, W1, b1r, W2p, b2p, W2w, b2w)
    return out_pos, out_w


# speedup vs baseline: 1.0013x; 1.0013x over previous
import functools

import jax
import jax.numpy as jnp
from jax.experimental import pallas as pl
from jax.experimental.pallas import tpu as pltpu

POS_DIM = 3
FEAT_DIM = 128
HIDDEN = 256
BLOCK_N = 2048


def _mlp_block_kernel(pos_ref, w_ref, w1_ref, b1_ref, w2p_ref, b2p_ref,
                      w2w_ref, b2w_ref, out_pos_ref, out_w_ref):
    w = w_ref[...]
    h = jnp.maximum(
        jnp.dot(w.astype(jnp.bfloat16), w1_ref[...],
                preferred_element_type=jnp.float32)
        + b1_ref[...], 0.0)
    hb = h.astype(jnp.bfloat16)
    dp = jnp.dot(hb, w2p_ref[...], preferred_element_type=jnp.float32)
    dw = jnp.dot(hb, w2w_ref[...], preferred_element_type=jnp.float32)
    out_pos_ref[...] = pos_ref[...] + dp + b2p_ref[...]
    out_w_ref[...] = w + dw + b2w_ref[...]


@functools.partial(jax.jit, static_argnames=())
def kernel(positions, weights, batch, W1, b1, W2, b2):
    del batch
    n = weights.shape[0]
    grid = (n // BLOCK_N,)

    W1 = W1.astype(jnp.bfloat16)
    W2p = W2[:, :POS_DIM].astype(jnp.bfloat16)
    W2w = W2[:, POS_DIM:].astype(jnp.bfloat16)
    b1r = b1.reshape(1, HIDDEN)
    b2p = b2[:POS_DIM].reshape(1, POS_DIM)
    b2w = b2[POS_DIM:].reshape(1, FEAT_DIM)

    row_block = lambda i: (i, 0)
    rep = lambda i: (0, 0)
    out_pos, out_w = pl.pallas_call(
        _mlp_block_kernel,
        grid=grid,
        in_specs=[
            pl.BlockSpec((BLOCK_N, POS_DIM), row_block),
            pl.BlockSpec((BLOCK_N, FEAT_DIM), row_block),
            pl.BlockSpec((FEAT_DIM, HIDDEN), rep),
            pl.BlockSpec((1, HIDDEN), rep),
            pl.BlockSpec((HIDDEN, POS_DIM), rep),
            pl.BlockSpec((1, POS_DIM), rep),
            pl.BlockSpec((HIDDEN, FEAT_DIM), rep),
            pl.BlockSpec((1, FEAT_DIM), rep),
        ],
        out_specs=[
            pl.BlockSpec((BLOCK_N, POS_DIM), row_block),
            pl.BlockSpec((BLOCK_N, FEAT_DIM), row_block),
        ],
        out_shape=[
            jax.ShapeDtypeStruct((n, POS_DIM), jnp.float32),
            jax.ShapeDtypeStruct((n, FEAT_DIM), jnp.float32),
        ],
        compiler_params=pltpu.CompilerParams(
            dimension_semantics=("parallel",),
        ),
    )(positions, weights, W1, b1r, W2p, b2p, W2w, b2w)
    return out_pos, out_w


# speedup vs baseline: 1.1267x; 1.1253x over previous
import functools

import jax
import jax.numpy as jnp
from jax.experimental import pallas as pl
from jax.experimental.pallas import tpu as pltpu

POS_DIM = 3
FEAT_DIM = 128
HIDDEN = 256
BLOCK_N = 8192


def _mlp_block_kernel(pos_ref, w_ref, w1_ref, b1_ref, w2p_ref, b2p_ref,
                      w2w_ref, b2w_ref, out_pos_ref, out_w_ref):
    w = w_ref[...]
    h = jnp.maximum(
        jnp.dot(w.astype(jnp.bfloat16), w1_ref[...],
                preferred_element_type=jnp.float32)
        + b1_ref[...], 0.0)
    hb = h.astype(jnp.bfloat16)
    dp = jnp.dot(hb, w2p_ref[...], preferred_element_type=jnp.float32)
    dw = jnp.dot(hb, w2w_ref[...], preferred_element_type=jnp.float32)
    out_pos_ref[...] = pos_ref[...] + dp + b2p_ref[...]
    out_w_ref[...] = w + dw + b2w_ref[...]


@functools.partial(jax.jit, static_argnames=())
def kernel(positions, weights, batch, W1, b1, W2, b2):
    del batch
    n = weights.shape[0]
    grid = (n // BLOCK_N,)

    W1 = W1.astype(jnp.bfloat16)
    W2p = W2[:, :POS_DIM].astype(jnp.bfloat16)
    W2w = W2[:, POS_DIM:].astype(jnp.bfloat16)
    b1r = b1.reshape(1, HIDDEN)
    b2p = b2[:POS_DIM].reshape(1, POS_DIM)
    b2w = b2[POS_DIM:].reshape(1, FEAT_DIM)

    row_block = lambda i: (i, 0)
    rep = lambda i: (0, 0)
    out_pos, out_w = pl.pallas_call(
        _mlp_block_kernel,
        grid=grid,
        in_specs=[
            pl.BlockSpec((BLOCK_N, POS_DIM), row_block),
            pl.BlockSpec((BLOCK_N, FEAT_DIM), row_block),
            pl.BlockSpec((FEAT_DIM, HIDDEN), rep),
            pl.BlockSpec((1, HIDDEN), rep),
            pl.BlockSpec((HIDDEN, POS_DIM), rep),
            pl.BlockSpec((1, POS_DIM), rep),
            pl.BlockSpec((HIDDEN, FEAT_DIM), rep),
            pl.BlockSpec((1, FEAT_DIM), rep),
        ],
        out_specs=[
            pl.BlockSpec((BLOCK_N, POS_DIM), row_block),
            pl.BlockSpec((BLOCK_N, FEAT_DIM), row_block),
        ],
        out_shape=[
            jax.ShapeDtypeStruct((n, POS_DIM), jnp.float32),
            jax.ShapeDtypeStruct((n, FEAT_DIM), jnp.float32),
        ],
        compiler_params=pltpu.CompilerParams(
            dimension_semantics=("parallel",),
        ),
    )(positions, weights, W1, b1r, W2p, b2p, W2w, b2w)
    return out_pos, out_w
